# baseline (speedup 1.0000x reference)
import functools

import jax
import jax.numpy as jnp
from jax import lax
from jax.experimental import pallas as pl
from jax.experimental.pallas import tpu as pltpu
from jax.experimental.pallas import tpu_sc as plsc

NUM_SAMPLES = 100000
NUM_FEATURES = 64
NUM_CLASSES = 1000
B = 64
TEMP = 0.05

NC = 2
NS = 16
NW = NC * NS
CPAD = 1024
RB = 8
NQ = 4
LQ = 1024
NFULL = (NUM_SAMPLES // LQ) // 8 * 8
SLOTS = NFULL // NQ
TAIL_BASE = NFULL * LQ
TAIL_LEN = (NUM_SAMPLES - TAIL_BASE) // 128 * 128
TAIL_GROUPS = TAIL_LEN // 16
assert NUM_SAMPLES - TAIL_BASE - TAIL_LEN == 32
TAIL2_BASE = TAIL_BASE + TAIL_LEN
TAIL2_LEN = NUM_SAMPLES - TAIL2_BASE
GROUPS = LQ // 16
NG = NUM_SAMPLES // 16
CG_BASE = NG // NW
CG_EXTRA = NG - CG_BASE * NW
CG_WIN = CG_BASE + 1
OUT_ROWS = 128


def _sc_body(featT_hbm, tailT_hbm, lab_hbm, idx_hbm,
             psum_hbm, tgt_hbm,
             lab0, fv0, lab1, fv1,
             clab, acc, hist, idxv, tgtv,
             ft32, lt32,
             sem0, sem1, gsem):
    c = lax.axis_index("c")
    s = lax.axis_index("s")
    wid = c * NS + s
    r_blk = wid // NQ
    q = wid % NQ
    row0 = pl.multiple_of(RB * r_blk, RB)

    zero16 = jnp.zeros((16,), jnp.float32)
    one16 = jnp.ones((16,), jnp.float32)
    zero16i = jnp.zeros((16,), jnp.int32)
    rvecs = [jnp.full((16,), r, jnp.int32) for r in range(RB)]

    def zbody(i, carry):
        for r in range(RB):
            acc[r, pl.ds(i * 16, 16)] = zero16
            hist[r, pl.ds(i * 16, 16)] = zero16
        return carry

    lax.fori_loop(0, CPAD // 16, zbody, 0)

    def fire(slot, labv, fv, sem):
        base = pl.multiple_of((q + NQ * slot) * LQ, LQ)
        pltpu.async_copy(lab_hbm.at[pl.ds(base, LQ)], labv, sem)
        pltpu.async_copy(
            featT_hbm.at[pl.ds(row0, RB), pl.ds(base, LQ)], fv, sem)

    def drain(labv, fv, sem):
        pltpu.make_async_copy(lab_hbm.at[pl.ds(0, LQ)], labv, sem).wait()
        pltpu.make_async_copy(
            featT_hbm.at[pl.ds(0, RB), pl.ds(0, LQ)], fv, sem).wait()

    def process(labv, fv, n_groups, unroll=16):
        @functools.partial(plsc.parallel_loop, 0, n_groups, unroll=unroll)
        def _body(g):
            sl = pl.ds(g * 16, 16)
            lab16 = labv[sl]
            for r in range(RB):
                plsc.addupdate_scatter(acc, [rvecs[r], lab16], fv[r, sl])

    bufs = ((lab0, fv0, sem0), (lab1, fv1, sem1))
    fire(0, *bufs[0])
    fire(1, *bufs[1])

    def outer(o, carry):
        for b in range(2):
            slot = 2 * o + b
            labv, fv, sem = bufs[b]
            drain(labv, fv, sem)
            process(labv, fv, GROUPS)

            @pl.when(slot + 2 < SLOTS)
            def _():
                fire(slot + 2, labv, fv, sem)
        return carry

    lax.fori_loop(0, SLOTS // 2, outer, 0)

    @pl.when(q == NQ - 1)
    def _tail():
        labv, fv, sem = bufs[0]
        pltpu.async_copy(lab_hbm.at[pl.ds(TAIL_BASE, TAIL_LEN)],
                         labv.at[pl.ds(0, TAIL_LEN)], sem)
        pltpu.async_copy(
            featT_hbm.at[pl.ds(row0, RB), pl.ds(TAIL_BASE, TAIL_LEN)],
            fv.at[:, pl.ds(0, TAIL_LEN)], sem)
        pltpu.make_async_copy(lab_hbm.at[pl.ds(0, TAIL_LEN)],
                              labv.at[pl.ds(0, TAIL_LEN)], sem).wait()
        pltpu.make_async_copy(
            featT_hbm.at[pl.ds(0, RB), pl.ds(0, TAIL_LEN)],
            fv.at[:, pl.ds(0, TAIL_LEN)], sem).wait()
        process(labv, fv, TAIL_GROUPS)

    @pl.when(q == 0)
    def _tail2():
        pltpu.sync_copy(lab_hbm.at[pl.ds(TAIL2_BASE, TAIL2_LEN)], lt32)
        pltpu.sync_copy(tailT_hbm.at[pl.ds(row0, RB), :], ft32)
        process(lt32, ft32, TAIL2_LEN // 16, unroll=1)

    g0 = CG_BASE * wid + jnp.minimum(wid, CG_EXTRA)
    g1 = CG_BASE * (wid + 1) + jnp.minimum(wid + 1, CG_EXTRA)
    sw = jnp.minimum(g0, NG - CG_WIN)
    pltpu.sync_copy(lab_hbm.at[pl.ds(sw * 16, CG_WIN * 16)], clab)

    @functools.partial(plsc.parallel_loop, 0, CG_WIN, unroll=4)
    def _cbody(j):
        gg = sw + j
        ok = jnp.logical_and(gg >= g0, gg < g1)
        m16 = jnp.broadcast_to(ok, (16,))
        lab16 = clab[pl.ds(j * 16, 16)]
        plsc.addupdate_scatter(hist, [zero16i, lab16], one16, mask=m16)

    pltpu.sync_copy(acc, psum_hbm.at[q, pl.ds(row0, RB), :])
    hrow0 = pl.multiple_of(NUM_FEATURES + RB * r_blk, RB)
    pltpu.sync_copy(hist, psum_hbm.at[q, pl.ds(hrow0, RB), :])

    @pl.when(jnp.logical_and(c == 0, s == 0))
    def _tgt():
        pltpu.sync_copy(idx_hbm, idxv)
        pltpu.async_copy(lab_hbm.at[idxv], tgtv, gsem).wait()
        pltpu.sync_copy(tgtv, tgt_hbm)


@functools.cache
def _make_sc_segment_sums():
  return pl.kernel(
    _sc_body,
    out_type=(
        jax.ShapeDtypeStruct((NQ, OUT_ROWS, CPAD), jnp.float32),
        jax.ShapeDtypeStruct((B,), jnp.int32),
    ),
    mesh=plsc.VectorSubcoreMesh(core_axis_name="c", subcore_axis_name="s"),
    compiler_params=pltpu.CompilerParams(needs_layout_passes=False),
    scratch_types=[
        pltpu.VMEM((LQ,), jnp.int32),
        pltpu.VMEM((RB, LQ), jnp.float32),
        pltpu.VMEM((LQ,), jnp.int32),
        pltpu.VMEM((RB, LQ), jnp.float32),
        pltpu.VMEM((CG_WIN * 16,), jnp.int32),
        pltpu.VMEM((RB, CPAD), jnp.float32),
        pltpu.VMEM((RB, CPAD), jnp.float32),
        pltpu.VMEM((B,), jnp.int32),
        pltpu.VMEM((B,), jnp.int32),
        pltpu.VMEM((RB, TAIL2_LEN), jnp.float32),
        pltpu.VMEM((TAIL2_LEN,), jnp.int32),
        pltpu.SemaphoreType.DMA,
        pltpu.SemaphoreType.DMA,
        pltpu.SemaphoreType.DMA,
    ],
  )


def _tc_body(psum_ref, inputs_ref, another_ref, tgt_ref, out_ref):
    psum = (psum_ref[0] + psum_ref[1]) + (psum_ref[2] + psum_ref[3])
    n = psum[NUM_FEATURES:NUM_FEATURES + 1, :]
    for r in range(1, RB):
        n = n + psum[NUM_FEATURES + RB * r:NUM_FEATURES + RB * r + 1, :]
    mask = (n > 0.0).astype(jnp.float32)
    mdT = psum[0:NUM_FEATURES, :] / (mask * n + (1.0 - mask))

    sim = lax.dot_general(
        inputs_ref[...], mdT, (((1,), (0,)), ((), ())),
        preferred_element_type=jnp.float32,
        precision=lax.Precision.HIGHEST) * (1.0 / TEMP)

    e = jnp.exp(sim) * mask
    denom = jnp.sum(e, axis=1, keepdims=True) + 1e-6
    cls = lax.broadcasted_iota(jnp.int32, (B, CPAD), 1)
    onehot = (cls == tgt_ref[...]).astype(jnp.float32)
    p = jnp.sum(e * onehot, axis=1, keepdims=True) / denom
    floss = jnp.sum(-((1.0 - p) ** 4) * jnp.log(p + 1e-6)) / B

    inp = lax.dot_general(
        onehot, mdT, (((1,), (1,)), ((), ())),
        preferred_element_type=jnp.float32,
        precision=lax.Precision.HIGHEST)
    inp = inp / jnp.sqrt(jnp.sum(inp * inp, axis=1, keepdims=True))
    another = another_ref[...]
    another = another / jnp.sqrt(jnp.sum(another * another, axis=1, keepdims=True))
    cml = -jnp.sum(inp * another) / B

    ninp = inputs_ref[...]
    ninp = ninp / jnp.sqrt(jnp.sum(ninp * ninp, axis=1, keepdims=True))
    cl = -jnp.sum(ninp * another) / B

    full = floss + cml + cl
    lane = lax.broadcasted_iota(jnp.int32, (1, 128), 1)
    out_ref[...] = (jnp.where(lane == 0, floss, 0.0)
                    + jnp.where(lane == 1, full, 0.0))


_tc_epilogue = pl.pallas_call(
    _tc_body,
    out_shape=jax.ShapeDtypeStruct((1, 128), jnp.float32),
)


def kernel(inputs, inputs_mask, another_inputs_full, indexes, back,
           features, labels, label_weight, label_count):
    del inputs_mask, label_weight, label_count
    featT = jnp.transpose(features)
    psum, tgt = _make_sc_segment_sums()(
        featT, lax.slice(featT, (0, TAIL2_BASE), (NUM_FEATURES, NUM_SAMPLES)),
        labels.astype(jnp.int32), indexes.astype(jnp.int32))
    out = _tc_epilogue(psum, inputs, another_inputs_full, tgt.reshape(B, 1))
    floss = out[0, 0]
    full_loss = out[0, 1]
    return jnp.where(back == 0, floss, full_loss)


# speedup vs baseline: 1.0511x; 1.0511x over previous
import functools

import jax
import jax.numpy as jnp
from jax import lax
from jax.experimental import pallas as pl
from jax.experimental.pallas import tpu as pltpu
from jax.experimental.pallas import tpu_sc as plsc

NUM_SAMPLES = 100000
NUM_FEATURES = 64
NUM_CLASSES = 1000
B = 64
TEMP = 0.05

NC = 2
NS = 16
NW = NC * NS
CPAD = 1024
RB = 8
NQ = 4
LQ = 2048
NFULL = (NUM_SAMPLES // LQ) // 8 * 8
SLOTS = NFULL // NQ
TAIL_BASE = NFULL * LQ
TAIL_LEN = (NUM_SAMPLES - TAIL_BASE) // 128 * 128
TAIL_GROUPS = TAIL_LEN // 16
assert NUM_SAMPLES - TAIL_BASE - TAIL_LEN == 32
TAIL2_BASE = TAIL_BASE + TAIL_LEN
TAIL2_LEN = NUM_SAMPLES - TAIL2_BASE
GROUPS = LQ // 16
NG = NUM_SAMPLES // 16
CG_BASE = NG // NW
CG_EXTRA = NG - CG_BASE * NW
CG_WIN = CG_BASE + 1
OUT_ROWS = 128


def _sc_body(featT_hbm, tailT_hbm, lab_hbm, idx_hbm,
             psum_hbm, tgt_hbm,
             lab0, fv0, lab1, fv1,
             clab, acc, hist, idxv, tgtv,
             ft32, lt32,
             sem0, sem1, gsem):
    c = lax.axis_index("c")
    s = lax.axis_index("s")
    wid = c * NS + s
    r_blk = wid // NQ
    q = wid % NQ
    row0 = pl.multiple_of(RB * r_blk, RB)

    zero16 = jnp.zeros((16,), jnp.float32)
    one16 = jnp.ones((16,), jnp.float32)
    zero16i = jnp.zeros((16,), jnp.int32)
    rvecs = [jnp.full((16,), r, jnp.int32) for r in range(RB)]

    def zbody(i, carry):
        for r in range(RB):
            acc[r, pl.ds(i * 16, 16)] = zero16
            hist[r, pl.ds(i * 16, 16)] = zero16
        return carry

    lax.fori_loop(0, CPAD // 16, zbody, 0)

    def fire(slot, labv, fv, sem):
        base = pl.multiple_of((q + NQ * slot) * LQ, LQ)
        pltpu.async_copy(lab_hbm.at[pl.ds(base, LQ)], labv, sem)
        pltpu.async_copy(
            featT_hbm.at[pl.ds(row0, RB), pl.ds(base, LQ)], fv, sem)

    def drain(labv, fv, sem):
        pltpu.make_async_copy(lab_hbm.at[pl.ds(0, LQ)], labv, sem).wait()
        pltpu.make_async_copy(
            featT_hbm.at[pl.ds(0, RB), pl.ds(0, LQ)], fv, sem).wait()

    def process(labv, fv, n_groups, unroll=16):
        @functools.partial(plsc.parallel_loop, 0, n_groups, unroll=unroll)
        def _body(g):
            sl = pl.ds(g * 16, 16)
            lab16 = labv[sl]
            for r in range(RB):
                plsc.addupdate_scatter(acc, [rvecs[r], lab16], fv[r, sl])

    bufs = ((lab0, fv0, sem0), (lab1, fv1, sem1))
    fire(0, *bufs[0])
    fire(1, *bufs[1])

    def outer(o, carry):
        for b in range(2):
            slot = 2 * o + b
            labv, fv, sem = bufs[b]
            drain(labv, fv, sem)
            process(labv, fv, GROUPS)

            @pl.when(slot + 2 < SLOTS)
            def _():
                fire(slot + 2, labv, fv, sem)
        return carry

    lax.fori_loop(0, SLOTS // 2, outer, 0)

    @pl.when(q == NQ - 1)
    def _tail():
        labv, fv, sem = bufs[0]
        pltpu.async_copy(lab_hbm.at[pl.ds(TAIL_BASE, TAIL_LEN)],
                         labv.at[pl.ds(0, TAIL_LEN)], sem)
        pltpu.async_copy(
            featT_hbm.at[pl.ds(row0, RB), pl.ds(TAIL_BASE, TAIL_LEN)],
            fv.at[:, pl.ds(0, TAIL_LEN)], sem)
        pltpu.make_async_copy(lab_hbm.at[pl.ds(0, TAIL_LEN)],
                              labv.at[pl.ds(0, TAIL_LEN)], sem).wait()
        pltpu.make_async_copy(
            featT_hbm.at[pl.ds(0, RB), pl.ds(0, TAIL_LEN)],
            fv.at[:, pl.ds(0, TAIL_LEN)], sem).wait()
        process(labv, fv, TAIL_GROUPS)

    @pl.when(q == 0)
    def _tail2():
        pltpu.sync_copy(lab_hbm.at[pl.ds(TAIL2_BASE, TAIL2_LEN)], lt32)
        pltpu.sync_copy(tailT_hbm.at[pl.ds(row0, RB), :], ft32)
        process(lt32, ft32, TAIL2_LEN // 16, unroll=1)

    g0 = CG_BASE * wid + jnp.minimum(wid, CG_EXTRA)
    g1 = CG_BASE * (wid + 1) + jnp.minimum(wid + 1, CG_EXTRA)
    sw = jnp.minimum(g0, NG - CG_WIN)
    pltpu.sync_copy(lab_hbm.at[pl.ds(sw * 16, CG_WIN * 16)], clab)

    @functools.partial(plsc.parallel_loop, 0, CG_WIN, unroll=4)
    def _cbody(j):
        gg = sw + j
        ok = jnp.logical_and(gg >= g0, gg < g1)
        m16 = jnp.broadcast_to(ok, (16,))
        lab16 = clab[pl.ds(j * 16, 16)]
        plsc.addupdate_scatter(hist, [zero16i, lab16], one16, mask=m16)

    pltpu.sync_copy(acc, psum_hbm.at[q, pl.ds(row0, RB), :])
    hrow0 = pl.multiple_of(NUM_FEATURES + RB * r_blk, RB)
    pltpu.sync_copy(hist, psum_hbm.at[q, pl.ds(hrow0, RB), :])

    @pl.when(jnp.logical_and(c == 0, s == 0))
    def _tgt():
        pltpu.sync_copy(idx_hbm, idxv)
        pltpu.async_copy(lab_hbm.at[idxv], tgtv, gsem).wait()
        pltpu.sync_copy(tgtv, tgt_hbm)


@functools.cache
def _make_sc_segment_sums():
  return pl.kernel(
    _sc_body,
    out_type=(
        jax.ShapeDtypeStruct((NQ, OUT_ROWS, CPAD), jnp.float32),
        jax.ShapeDtypeStruct((B,), jnp.int32),
    ),
    mesh=plsc.VectorSubcoreMesh(core_axis_name="c", subcore_axis_name="s"),
    compiler_params=pltpu.CompilerParams(needs_layout_passes=False),
    scratch_types=[
        pltpu.VMEM((LQ,), jnp.int32),
        pltpu.VMEM((RB, LQ), jnp.float32),
        pltpu.VMEM((LQ,), jnp.int32),
        pltpu.VMEM((RB, LQ), jnp.float32),
        pltpu.VMEM((CG_WIN * 16,), jnp.int32),
        pltpu.VMEM((RB, CPAD), jnp.float32),
        pltpu.VMEM((RB, CPAD), jnp.float32),
        pltpu.VMEM((B,), jnp.int32),
        pltpu.VMEM((B,), jnp.int32),
        pltpu.VMEM((RB, TAIL2_LEN), jnp.float32),
        pltpu.VMEM((TAIL2_LEN,), jnp.int32),
        pltpu.SemaphoreType.DMA,
        pltpu.SemaphoreType.DMA,
        pltpu.SemaphoreType.DMA,
    ],
  )


def _tc_body(psum_ref, inputs_ref, another_ref, tgt_ref, out_ref):
    psum = (psum_ref[0] + psum_ref[1]) + (psum_ref[2] + psum_ref[3])
    n = psum[NUM_FEATURES:NUM_FEATURES + 1, :]
    for r in range(1, RB):
        n = n + psum[NUM_FEATURES + RB * r:NUM_FEATURES + RB * r + 1, :]
    mask = (n > 0.0).astype(jnp.float32)
    mdT = psum[0:NUM_FEATURES, :] / (mask * n + (1.0 - mask))

    sim = lax.dot_general(
        inputs_ref[...], mdT, (((1,), (0,)), ((), ())),
        preferred_element_type=jnp.float32,
        precision=lax.Precision.HIGHEST) * (1.0 / TEMP)

    e = jnp.exp(sim) * mask
    denom = jnp.sum(e, axis=1, keepdims=True) + 1e-6
    cls = lax.broadcasted_iota(jnp.int32, (B, CPAD), 1)
    onehot = (cls == tgt_ref[...]).astype(jnp.float32)
    p = jnp.sum(e * onehot, axis=1, keepdims=True) / denom
    floss = jnp.sum(-((1.0 - p) ** 4) * jnp.log(p + 1e-6)) / B

    inp = lax.dot_general(
        onehot, mdT, (((1,), (1,)), ((), ())),
        preferred_element_type=jnp.float32,
        precision=lax.Precision.HIGHEST)
    inp = inp / jnp.sqrt(jnp.sum(inp * inp, axis=1, keepdims=True))
    another = another_ref[...]
    another = another / jnp.sqrt(jnp.sum(another * another, axis=1, keepdims=True))
    cml = -jnp.sum(inp * another) / B

    ninp = inputs_ref[...]
    ninp = ninp / jnp.sqrt(jnp.sum(ninp * ninp, axis=1, keepdims=True))
    cl = -jnp.sum(ninp * another) / B

    full = floss + cml + cl
    lane = lax.broadcasted_iota(jnp.int32, (1, 128), 1)
    out_ref[...] = (jnp.where(lane == 0, floss, 0.0)
                    + jnp.where(lane == 1, full, 0.0))


_tc_epilogue = pl.pallas_call(
    _tc_body,
    out_shape=jax.ShapeDtypeStruct((1, 128), jnp.float32),
)


def kernel(inputs, inputs_mask, another_inputs_full, indexes, back,
           features, labels, label_weight, label_count):
    del inputs_mask, label_weight, label_count
    featT = jnp.transpose(features)
    psum, tgt = _make_sc_segment_sums()(
        featT, lax.slice(featT, (0, TAIL2_BASE), (NUM_FEATURES, NUM_SAMPLES)),
        labels.astype(jnp.int32), indexes.astype(jnp.int32))
    out = _tc_epilogue(psum, inputs, another_inputs_full, tgt.reshape(B, 1))
    floss = out[0, 0]
    full_loss = out[0, 1]
    return jnp.where(back == 0, floss, full_loss)


# speedup vs baseline: 1.0977x; 1.0444x over previous
import functools

import jax
import jax.numpy as jnp
from jax import lax
from jax.experimental import pallas as pl
from jax.experimental.pallas import tpu as pltpu
from jax.experimental.pallas import tpu_sc as plsc

NUM_SAMPLES = 100000
NUM_FEATURES = 64
NUM_CLASSES = 1000
B = 64
TEMP = 0.05

NC = 2
NS = 16
NW = NC * NS
CPAD = 1024
RB = 8
NQ = 4
LQ = 2048
NFULL = (NUM_SAMPLES // LQ) // 8 * 8
SLOTS = NFULL // NQ
TAIL_BASE = NFULL * LQ
TAIL_LEN = (NUM_SAMPLES - TAIL_BASE) // 128 * 128
TAIL_GROUPS = TAIL_LEN // 16
assert NUM_SAMPLES - TAIL_BASE - TAIL_LEN == 32
TAIL2_BASE = TAIL_BASE + TAIL_LEN
TAIL2_LEN = NUM_SAMPLES - TAIL2_BASE
GROUPS = LQ // 16
NG = NUM_SAMPLES // 16
CG_BASE = NG // NW
CG_EXTRA = NG - CG_BASE * NW
CG_WIN = CG_BASE + 1
OUT_ROWS = 128


def _sc_body(featT_hbm, tailT_hbm, lab_hbm, idx_hbm,
             psum_hbm, tgt_hbm,
             lab0, fv0, lab1, fv1,
             clab, acc, hist, idxv, tgtv,
             ft32, lt32,
             sem0, sem1, gsem, tsem):
    c = lax.axis_index("c")
    s = lax.axis_index("s")
    wid = c * NS + s
    r_blk = wid // NQ
    q = wid % NQ
    row0 = pl.multiple_of(RB * r_blk, RB)

    zero16 = jnp.zeros((16,), jnp.float32)
    one16 = jnp.ones((16,), jnp.float32)
    zero16i = jnp.zeros((16,), jnp.int32)
    rvecs = [jnp.full((16,), r, jnp.int32) for r in range(RB)]

    def fire(slot, labv, fv, sem):
        base = pl.multiple_of((q + NQ * slot) * LQ, LQ)
        pltpu.async_copy(lab_hbm.at[pl.ds(base, LQ)], labv, sem)
        pltpu.async_copy(
            featT_hbm.at[pl.ds(row0, RB), pl.ds(base, LQ)], fv, sem)

    def drain(labv, fv, sem):
        pltpu.make_async_copy(lab_hbm.at[pl.ds(0, LQ)], labv, sem).wait()
        pltpu.make_async_copy(
            featT_hbm.at[pl.ds(0, RB), pl.ds(0, LQ)], fv, sem).wait()

    def process(labv, fv, n_groups, unroll=16):
        @functools.partial(plsc.parallel_loop, 0, n_groups, unroll=unroll)
        def _body(g):
            sl = pl.ds(g * 16, 16)
            lab16 = labv[sl]
            for r in range(RB):
                plsc.addupdate_scatter(acc, [rvecs[r], lab16], fv[r, sl])

    bufs = ((lab0, fv0, sem0), (lab1, fv1, sem1))
    fire(0, *bufs[0])
    fire(1, *bufs[1])

    g0 = CG_BASE * wid + jnp.minimum(wid, CG_EXTRA)
    g1 = CG_BASE * (wid + 1) + jnp.minimum(wid + 1, CG_EXTRA)
    sw = jnp.minimum(g0, NG - CG_WIN)
    pltpu.async_copy(lab_hbm.at[pl.ds(sw * 16, CG_WIN * 16)], clab, gsem)

    @pl.when(jnp.logical_and(c == 0, s == 0))
    def _tgt():
        pltpu.sync_copy(idx_hbm, idxv)
        pltpu.async_copy(lab_hbm.at[idxv], tgtv, tsem).wait()
        pltpu.sync_copy(tgtv, tgt_hbm)

    def zbody(i, carry):
        for r in range(RB):
            acc[r, pl.ds(i * 16, 16)] = zero16
        hist[0, pl.ds(i * 16, 16)] = zero16
        return carry

    lax.fori_loop(0, CPAD // 16, zbody, 0)

    def outer(o, carry):
        for b in range(2):
            slot = 2 * o + b
            labv, fv, sem = bufs[b]
            drain(labv, fv, sem)
            process(labv, fv, GROUPS)

            @pl.when(slot + 2 < SLOTS)
            def _():
                fire(slot + 2, labv, fv, sem)
        return carry

    lax.fori_loop(0, SLOTS // 2, outer, 0)

    @pl.when(q == NQ - 1)
    def _tail():
        labv, fv, sem = bufs[0]
        pltpu.async_copy(lab_hbm.at[pl.ds(TAIL_BASE, TAIL_LEN)],
                         labv.at[pl.ds(0, TAIL_LEN)], sem)
        pltpu.async_copy(
            featT_hbm.at[pl.ds(row0, RB), pl.ds(TAIL_BASE, TAIL_LEN)],
            fv.at[:, pl.ds(0, TAIL_LEN)], sem)
        pltpu.make_async_copy(lab_hbm.at[pl.ds(0, TAIL_LEN)],
                              labv.at[pl.ds(0, TAIL_LEN)], sem).wait()
        pltpu.make_async_copy(
            featT_hbm.at[pl.ds(0, RB), pl.ds(0, TAIL_LEN)],
            fv.at[:, pl.ds(0, TAIL_LEN)], sem).wait()
        process(labv, fv, TAIL_GROUPS)

    @pl.when(q == 0)
    def _tail2():
        pltpu.sync_copy(lab_hbm.at[pl.ds(TAIL2_BASE, TAIL2_LEN)], lt32)
        pltpu.sync_copy(tailT_hbm.at[pl.ds(row0, RB), :], ft32)
        process(lt32, ft32, TAIL2_LEN // 16, unroll=1)

    pltpu.make_async_copy(
        lab_hbm.at[pl.ds(0, CG_WIN * 16)], clab, gsem).wait()

    @functools.partial(plsc.parallel_loop, 0, CG_WIN, unroll=4)
    def _cbody(j):
        gg = sw + j
        ok = jnp.logical_and(gg >= g0, gg < g1)
        m16 = jnp.broadcast_to(ok, (16,))
        lab16 = clab[pl.ds(j * 16, 16)]
        plsc.addupdate_scatter(hist, [zero16i, lab16], one16, mask=m16)

    pltpu.sync_copy(acc, psum_hbm.at[q, pl.ds(row0, RB), :])
    hrow0 = pl.multiple_of(NUM_FEATURES + RB * r_blk, RB)
    pltpu.sync_copy(hist, psum_hbm.at[q, pl.ds(hrow0, RB), :])


@functools.cache
def _make_sc_segment_sums():
  return pl.kernel(
    _sc_body,
    out_type=(
        jax.ShapeDtypeStruct((NQ, OUT_ROWS, CPAD), jnp.float32),
        jax.ShapeDtypeStruct((B,), jnp.int32),
    ),
    mesh=plsc.VectorSubcoreMesh(core_axis_name="c", subcore_axis_name="s"),
    compiler_params=pltpu.CompilerParams(needs_layout_passes=False),
    scratch_types=[
        pltpu.VMEM((LQ,), jnp.int32),
        pltpu.VMEM((RB, LQ), jnp.float32),
        pltpu.VMEM((LQ,), jnp.int32),
        pltpu.VMEM((RB, LQ), jnp.float32),
        pltpu.VMEM((CG_WIN * 16,), jnp.int32),
        pltpu.VMEM((RB, CPAD), jnp.float32),
        pltpu.VMEM((RB, CPAD), jnp.float32),
        pltpu.VMEM((B,), jnp.int32),
        pltpu.VMEM((B,), jnp.int32),
        pltpu.VMEM((RB, TAIL2_LEN), jnp.float32),
        pltpu.VMEM((TAIL2_LEN,), jnp.int32),
        pltpu.SemaphoreType.DMA,
        pltpu.SemaphoreType.DMA,
        pltpu.SemaphoreType.DMA,
        pltpu.SemaphoreType.DMA,
    ],
  )


def _tc_body(psum_ref, inputs_ref, another_ref, tgt_ref, out_ref):
    psum = (psum_ref[0] + psum_ref[1]) + (psum_ref[2] + psum_ref[3])
    n = psum[NUM_FEATURES:NUM_FEATURES + 1, :]
    for r in range(1, RB):
        n = n + psum[NUM_FEATURES + RB * r:NUM_FEATURES + RB * r + 1, :]
    mask = (n > 0.0).astype(jnp.float32)
    mdT = psum[0:NUM_FEATURES, :] / (mask * n + (1.0 - mask))

    sim = lax.dot_general(
        inputs_ref[...], mdT, (((1,), (0,)), ((), ())),
        preferred_element_type=jnp.float32,
        precision=lax.Precision.HIGHEST) * (1.0 / TEMP)

    e = jnp.exp(sim) * mask
    denom = jnp.sum(e, axis=1, keepdims=True) + 1e-6
    cls = lax.broadcasted_iota(jnp.int32, (B, CPAD), 1)
    onehot = (cls == tgt_ref[...]).astype(jnp.float32)
    p = jnp.sum(e * onehot, axis=1, keepdims=True) / denom
    floss = jnp.sum(-((1.0 - p) ** 4) * jnp.log(p + 1e-6)) / B

    inp = lax.dot_general(
        onehot, mdT, (((1,), (1,)), ((), ())),
        preferred_element_type=jnp.float32,
        precision=lax.Precision.HIGHEST)
    inp = inp / jnp.sqrt(jnp.sum(inp * inp, axis=1, keepdims=True))
    another = another_ref[...]
    another = another / jnp.sqrt(jnp.sum(another * another, axis=1, keepdims=True))
    cml = -jnp.sum(inp * another) / B

    ninp = inputs_ref[...]
    ninp = ninp / jnp.sqrt(jnp.sum(ninp * ninp, axis=1, keepdims=True))
    cl = -jnp.sum(ninp * another) / B

    full = floss + cml + cl
    lane = lax.broadcasted_iota(jnp.int32, (1, 128), 1)
    out_ref[...] = (jnp.where(lane == 0, floss, 0.0)
                    + jnp.where(lane == 1, full, 0.0))


_tc_epilogue = pl.pallas_call(
    _tc_body,
    out_shape=jax.ShapeDtypeStruct((1, 128), jnp.float32),
)


def kernel(inputs, inputs_mask, another_inputs_full, indexes, back,
           features, labels, label_weight, label_count):
    del inputs_mask, label_weight, label_count
    featT = jnp.transpose(features)
    psum, tgt = _make_sc_segment_sums()(
        featT, lax.slice(featT, (0, TAIL2_BASE), (NUM_FEATURES, NUM_SAMPLES)),
        labels.astype(jnp.int32), indexes.astype(jnp.int32))
    out = _tc_epilogue(psum, inputs, another_inputs_full, tgt.reshape(B, 1))
    floss = out[0, 0]
    full_loss = out[0, 1]
    return jnp.where(back == 0, floss, full_loss)


# speedup vs baseline: 1.1103x; 1.0115x over previous
import functools

import jax
import jax.numpy as jnp
from jax import lax
from jax.experimental import pallas as pl
from jax.experimental.pallas import tpu as pltpu
from jax.experimental.pallas import tpu_sc as plsc

NUM_SAMPLES = 100000
NUM_FEATURES = 64
NUM_CLASSES = 1000
B = 64
TEMP = 0.05

NC = 2
NS = 16
NW = NC * NS
CPAD = 1024
RB = 8
NQ = 4
LQ = 2048
NFULL = (NUM_SAMPLES // LQ) // 8 * 8
SLOTS = NFULL // NQ
TAIL_BASE = NFULL * LQ
TAIL_LEN = (NUM_SAMPLES - TAIL_BASE) // 128 * 128
TAIL_GROUPS = TAIL_LEN // 16
assert NUM_SAMPLES - TAIL_BASE - TAIL_LEN == 32
TAIL2_BASE = TAIL_BASE + TAIL_LEN
TAIL2_LEN = NUM_SAMPLES - TAIL2_BASE
GROUPS = LQ // 16
NG = NUM_SAMPLES // 16
CG_BASE = NG // NW
CG_EXTRA = NG - CG_BASE * NW
CG_WIN = CG_BASE + 1
OUT_ROWS = 128


def _sc_body(featT_hbm, tailT_hbm, lab_hbm, idx_hbm,
             psum_hbm, tgt_hbm,
             lab0, fv0, lab1, fv1,
             clab, acc, hist, idxv, tgtv,
             ft32, lt32,
             sem0, sem1, gsem, tsem):
    c = lax.axis_index("c")
    s = lax.axis_index("s")
    wid = c * NS + s
    r_blk = wid // NQ
    q = wid % NQ
    row0 = pl.multiple_of(RB * r_blk, RB)

    zero16 = jnp.zeros((16,), jnp.float32)
    one16 = jnp.ones((16,), jnp.float32)
    zero16i = jnp.zeros((16,), jnp.int32)
    rvecs = [jnp.full((16,), r, jnp.int32) for r in range(RB)]

    def fire(slot, labv, fv, sem):
        base = pl.multiple_of((q + NQ * slot) * LQ, LQ)
        pltpu.async_copy(lab_hbm.at[pl.ds(base, LQ)], labv, sem)
        pltpu.async_copy(
            featT_hbm.at[pl.ds(row0, RB), pl.ds(base, LQ)], fv, sem)

    def drain(labv, fv, sem):
        pltpu.make_async_copy(lab_hbm.at[pl.ds(0, LQ)], labv, sem).wait()
        pltpu.make_async_copy(
            featT_hbm.at[pl.ds(0, RB), pl.ds(0, LQ)], fv, sem).wait()

    def process(labv, fv, n_groups, unroll=4):
        @functools.partial(plsc.parallel_loop, 0, n_groups, unroll=unroll)
        def _body(g):
            sl = pl.ds(g * 16, 16)
            lab16 = labv[sl]
            for r in range(RB):
                plsc.addupdate_scatter(acc, [rvecs[r], lab16], fv[r, sl])

    bufs = ((lab0, fv0, sem0), (lab1, fv1, sem1))
    fire(0, *bufs[0])
    fire(1, *bufs[1])

    g0 = CG_BASE * wid + jnp.minimum(wid, CG_EXTRA)
    g1 = CG_BASE * (wid + 1) + jnp.minimum(wid + 1, CG_EXTRA)
    sw = jnp.minimum(g0, NG - CG_WIN)
    pltpu.async_copy(lab_hbm.at[pl.ds(sw * 16, CG_WIN * 16)], clab, gsem)

    @pl.when(jnp.logical_and(c == 0, s == 0))
    def _tgt():
        pltpu.sync_copy(idx_hbm, idxv)
        pltpu.async_copy(lab_hbm.at[idxv], tgtv, tsem).wait()
        pltpu.sync_copy(tgtv, tgt_hbm)

    def zbody(i, carry):
        for r in range(RB):
            acc[r, pl.ds(i * 16, 16)] = zero16
        hist[0, pl.ds(i * 16, 16)] = zero16
        return carry

    lax.fori_loop(0, CPAD // 16, zbody, 0)

    def outer(o, carry):
        for b in range(2):
            slot = 2 * o + b
            labv, fv, sem = bufs[b]
            drain(labv, fv, sem)
            process(labv, fv, GROUPS)

            @pl.when(slot + 2 < SLOTS)
            def _():
                fire(slot + 2, labv, fv, sem)
        return carry

    lax.fori_loop(0, SLOTS // 2, outer, 0)

    @pl.when(q == NQ - 1)
    def _tail():
        labv, fv, sem = bufs[0]
        pltpu.async_copy(lab_hbm.at[pl.ds(TAIL_BASE, TAIL_LEN)],
                         labv.at[pl.ds(0, TAIL_LEN)], sem)
        pltpu.async_copy(
            featT_hbm.at[pl.ds(row0, RB), pl.ds(TAIL_BASE, TAIL_LEN)],
            fv.at[:, pl.ds(0, TAIL_LEN)], sem)
        pltpu.make_async_copy(lab_hbm.at[pl.ds(0, TAIL_LEN)],
                              labv.at[pl.ds(0, TAIL_LEN)], sem).wait()
        pltpu.make_async_copy(
            featT_hbm.at[pl.ds(0, RB), pl.ds(0, TAIL_LEN)],
            fv.at[:, pl.ds(0, TAIL_LEN)], sem).wait()
        process(labv, fv, TAIL_GROUPS)

    @pl.when(q == 0)
    def _tail2():
        pltpu.sync_copy(lab_hbm.at[pl.ds(TAIL2_BASE, TAIL2_LEN)], lt32)
        pltpu.sync_copy(tailT_hbm.at[pl.ds(row0, RB), :], ft32)
        process(lt32, ft32, TAIL2_LEN // 16, unroll=1)

    pltpu.make_async_copy(
        lab_hbm.at[pl.ds(0, CG_WIN * 16)], clab, gsem).wait()

    @functools.partial(plsc.parallel_loop, 0, CG_WIN, unroll=4)
    def _cbody(j):
        gg = sw + j
        ok = jnp.logical_and(gg >= g0, gg < g1)
        m16 = jnp.broadcast_to(ok, (16,))
        lab16 = clab[pl.ds(j * 16, 16)]
        plsc.addupdate_scatter(hist, [zero16i, lab16], one16, mask=m16)

    pltpu.sync_copy(acc, psum_hbm.at[q, pl.ds(row0, RB), :])
    hrow0 = pl.multiple_of(NUM_FEATURES + RB * r_blk, RB)
    pltpu.sync_copy(hist, psum_hbm.at[q, pl.ds(hrow0, RB), :])


@functools.cache
def _make_sc_segment_sums():
  return pl.kernel(
    _sc_body,
    out_type=(
        jax.ShapeDtypeStruct((NQ, OUT_ROWS, CPAD), jnp.float32),
        jax.ShapeDtypeStruct((B,), jnp.int32),
    ),
    mesh=plsc.VectorSubcoreMesh(core_axis_name="c", subcore_axis_name="s"),
    compiler_params=pltpu.CompilerParams(needs_layout_passes=False),
    scratch_types=[
        pltpu.VMEM((LQ,), jnp.int32),
        pltpu.VMEM((RB, LQ), jnp.float32),
        pltpu.VMEM((LQ,), jnp.int32),
        pltpu.VMEM((RB, LQ), jnp.float32),
        pltpu.VMEM((CG_WIN * 16,), jnp.int32),
        pltpu.VMEM((RB, CPAD), jnp.float32),
        pltpu.VMEM((RB, CPAD), jnp.float32),
        pltpu.VMEM((B,), jnp.int32),
        pltpu.VMEM((B,), jnp.int32),
        pltpu.VMEM((RB, TAIL2_LEN), jnp.float32),
        pltpu.VMEM((TAIL2_LEN,), jnp.int32),
        pltpu.SemaphoreType.DMA,
        pltpu.SemaphoreType.DMA,
        pltpu.SemaphoreType.DMA,
        pltpu.SemaphoreType.DMA,
    ],
  )


def _tc_body(psum_ref, inputs_ref, another_ref, tgt_ref, out_ref):
    psum = (psum_ref[0] + psum_ref[1]) + (psum_ref[2] + psum_ref[3])
    n = psum[NUM_FEATURES:NUM_FEATURES + 1, :]
    for r in range(1, RB):
        n = n + psum[NUM_FEATURES + RB * r:NUM_FEATURES + RB * r + 1, :]
    mask = (n > 0.0).astype(jnp.float32)
    mdT = psum[0:NUM_FEATURES, :] / (mask * n + (1.0 - mask))

    sim = lax.dot_general(
        inputs_ref[...], mdT, (((1,), (0,)), ((), ())),
        preferred_element_type=jnp.float32,
        precision=lax.Precision.HIGHEST) * (1.0 / TEMP)

    e = jnp.exp(sim) * mask
    denom = jnp.sum(e, axis=1, keepdims=True) + 1e-6
    cls = lax.broadcasted_iota(jnp.int32, (B, CPAD), 1)
    onehot = (cls == tgt_ref[...]).astype(jnp.float32)
    p = jnp.sum(e * onehot, axis=1, keepdims=True) / denom
    floss = jnp.sum(-((1.0 - p) ** 4) * jnp.log(p + 1e-6)) / B

    inp = lax.dot_general(
        onehot, mdT, (((1,), (1,)), ((), ())),
        preferred_element_type=jnp.float32,
        precision=lax.Precision.HIGHEST)
    inp = inp / jnp.sqrt(jnp.sum(inp * inp, axis=1, keepdims=True))
    another = another_ref[...]
    another = another / jnp.sqrt(jnp.sum(another * another, axis=1, keepdims=True))
    cml = -jnp.sum(inp * another) / B

    ninp = inputs_ref[...]
    ninp = ninp / jnp.sqrt(jnp.sum(ninp * ninp, axis=1, keepdims=True))
    cl = -jnp.sum(ninp * another) / B

    full = floss + cml + cl
    lane = lax.broadcasted_iota(jnp.int32, (1, 128), 1)
    out_ref[...] = (jnp.where(lane == 0, floss, 0.0)
                    + jnp.where(lane == 1, full, 0.0))


_tc_epilogue = pl.pallas_call(
    _tc_body,
    out_shape=jax.ShapeDtypeStruct((1, 128), jnp.float32),
)


def kernel(inputs, inputs_mask, another_inputs_full, indexes, back,
           features, labels, label_weight, label_count):
    del inputs_mask, label_weight, label_count
    featT = jnp.transpose(features)
    psum, tgt = _make_sc_segment_sums()(
        featT, lax.slice(featT, (0, TAIL2_BASE), (NUM_FEATURES, NUM_SAMPLES)),
        labels.astype(jnp.int32), indexes.astype(jnp.int32))
    out = _tc_epilogue(psum, inputs, another_inputs_full, tgt.reshape(B, 1))
    floss = out[0, 0]
    full_loss = out[0, 1]
    return jnp.where(back == 0, floss, full_loss)
